# Initial kernel scaffold; baseline (speedup 1.0000x reference)
import jax
import jax.numpy as jnp
from jax.experimental import pallas as pl


def kernel(x, init_adj, W1, b1, W2, b2, Wout, bout, W_lin, b_lin):
    raise NotImplementedError("write your pallas kernel here")


# speedup vs baseline: 1.1558x; 1.1558x over previous
import functools

import jax
import jax.numpy as jnp
from jax.experimental import pallas as pl
from jax.experimental.pallas import tpu as pltpu

B, N, F, H, O, R = 32, 512, 256, 128, 128, 53


def _gcn_kernel(x_ref, adj_ref, W1_ref, b1_ref, W2_ref, b2_ref,
                Wout_ref, bout_ref, Wlin_ref, blin_ref, out_ref):
    A = adj_ref[0]
    xb = x_ref[0]
    deg = jnp.sum(A, axis=1)
    dis = jax.lax.rsqrt(jnp.clip(deg, 1e-12, None))
    An = A * dis[:, None] * dis[None, :]

    xw = jnp.dot(xb, W1_ref[:, :], preferred_element_type=jnp.float32)
    h = jnp.maximum(
        jnp.dot(An, xw, preferred_element_type=jnp.float32) + b1_ref[:, :], 0.0)
    hw = jnp.dot(h, W2_ref[:, :], preferred_element_type=jnp.float32)
    h = jnp.maximum(
        jnp.dot(An, hw, preferred_element_type=jnp.float32) + b2_ref[:, :], 0.0)
    nw = jnp.dot(h, Wout_ref[:, :], preferred_element_type=jnp.float32)
    nv = jnp.dot(An, nw, preferred_element_type=jnp.float32) + bout_ref[:, :]

    ge = jnp.max(nv, axis=0, keepdims=True)
    out_ref[0, :, :] = (
        jnp.dot(ge, Wlin_ref[:, :], preferred_element_type=jnp.float32)
        + blin_ref[:, :])


@functools.partial(jax.jit, static_argnames=())
def kernel(x, init_adj, W1, b1, W2, b2, Wout, bout, W_lin, b_lin):
    b1r = b1.reshape(1, H)
    b2r = b2.reshape(1, H)
    boutr = bout.reshape(1, O)
    blinr = b_lin.reshape(1, R)
    grid = (B,)
    out = pl.pallas_call(
        _gcn_kernel,
        grid=grid,
        in_specs=[
            pl.BlockSpec((1, N, F), lambda b: (b, 0, 0)),
            pl.BlockSpec((1, N, N), lambda b: (b, 0, 0)),
            pl.BlockSpec((F, H), lambda b: (0, 0)),
            pl.BlockSpec((1, H), lambda b: (0, 0)),
            pl.BlockSpec((H, H), lambda b: (0, 0)),
            pl.BlockSpec((1, H), lambda b: (0, 0)),
            pl.BlockSpec((H, O), lambda b: (0, 0)),
            pl.BlockSpec((1, O), lambda b: (0, 0)),
            pl.BlockSpec((O, R), lambda b: (0, 0)),
            pl.BlockSpec((1, R), lambda b: (0, 0)),
        ],
        out_specs=pl.BlockSpec((1, 1, R), lambda b: (b, 0, 0)),
        out_shape=jax.ShapeDtypeStruct((B, 1, R), jnp.float32),
        compiler_params=pltpu.CompilerParams(
            dimension_semantics=("arbitrary",),
        ),
    )(x, init_adj, W1, b1r, W2, b2r, Wout, boutr, W_lin, blinr)
    return out.reshape(B, R)
